# baseline (speedup 1.0000x reference)
import functools

import jax
import jax.numpy as jnp
from jax import lax
from jax.experimental import pallas as pl
from jax.experimental.pallas import tpu as pltpu
from jax.experimental.pallas import tpu_sc as plsc

HID = 1024
B = 4
S = 2048
EPS = 1e-12
LANES = 16
NWORKERS = 32
TOK_PER_W = (B * S) // NWORKERS
CHUNK = 16
NCHUNK = TOK_PER_W // CHUNK
NSLICE = HID // LANES
INV_HID = 1.0 / HID


def _rsqrt_newton(v):
    i = plsc.bitcast(v, jnp.int32)
    i = 0x5F3759DF - lax.shift_right_logical(i, 1)
    y = plsc.bitcast(i, jnp.float32)
    half = v * 0.5
    for _ in range(3):
        y = y * (1.5 - half * y * y)
    return y


def _body(ids_hbm, seg_hbm, tok_tab, seg_tab, pos_tab, gamma_hbm, beta_hbm,
          out_hbm, idx_v, sidx_v, tok_buf, seg_buf, pos_buf, gamma_v, beta_v,
          sem_t, sem_s):
    wid = lax.axis_index("s") * 2 + lax.axis_index("c")
    wbase = wid * TOK_PER_W
    pos0 = lax.rem(wbase, S)

    pltpu.sync_copy(gamma_hbm, gamma_v)
    pltpu.sync_copy(beta_hbm, beta_v)

    def chunk_body(g, _):
        base = wbase + g * CHUNK
        pbase = pos0 + g * CHUNK
        pltpu.sync_copy(ids_hbm.at[pl.ds(base, CHUNK)], idx_v)
        pltpu.sync_copy(seg_hbm.at[pl.ds(base, CHUNK)], sidx_v)
        cp_t = pltpu.async_copy(tok_tab.at[idx_v], tok_buf, sem_t)
        cp_s = pltpu.async_copy(seg_tab.at[sidx_v], seg_buf, sem_s)
        pltpu.sync_copy(pos_tab.at[pl.ds(pbase, CHUNK)], pos_buf)
        cp_t.wait()
        cp_s.wait()

        def row_body(j, _):
            def sum_body(k, carry):
                s0, s1 = carry
                sl = pl.ds(k * LANES, LANES)
                x = tok_buf[j, sl] + seg_buf[j, sl] + pos_buf[j, sl]
                tok_buf[j, sl] = x
                return (s0 + x, s1 + x * x)

            zero = jnp.zeros((LANES,), jnp.float32)
            s0, s1 = lax.fori_loop(0, NSLICE, sum_body, (zero, zero))
            mean = jnp.sum(s0) * INV_HID
            var = jnp.sum(s1) * INV_HID - mean * mean
            rstd_v = _rsqrt_newton(jnp.full((LANES,), var + EPS, jnp.float32))
            mean_v = jnp.full((LANES,), mean, jnp.float32)

            def norm_body(k, _):
                sl = pl.ds(k * LANES, LANES)
                x = tok_buf[j, sl]
                tok_buf[j, sl] = (x - mean_v) * rstd_v * gamma_v[sl] + beta_v[sl]
                return 0

            lax.fori_loop(0, NSLICE, norm_body, 0)
            return 0

        lax.fori_loop(0, CHUNK, row_body, 0)
        pltpu.sync_copy(tok_buf, out_hbm.at[pl.ds(base, CHUNK)])
        return 0

    lax.fori_loop(0, NCHUNK, chunk_body, 0)


@functools.partial(jax.jit, static_argnums=())
def _embed_ln(ids, segs, tok_tab, seg_tab, pos_tab, gamma, beta):
    mesh = plsc.VectorSubcoreMesh(core_axis_name="c", subcore_axis_name="s")
    f = functools.partial(
        pl.kernel,
        out_type=jax.ShapeDtypeStruct((B * S, HID), jnp.float32),
        mesh=mesh,
        compiler_params=pltpu.CompilerParams(needs_layout_passes=False),
        scratch_types=[
            pltpu.VMEM((CHUNK,), jnp.int32),
            pltpu.VMEM((CHUNK,), jnp.int32),
            pltpu.VMEM((CHUNK, HID), jnp.float32),
            pltpu.VMEM((CHUNK, HID), jnp.float32),
            pltpu.VMEM((CHUNK, HID), jnp.float32),
            pltpu.VMEM((HID,), jnp.float32),
            pltpu.VMEM((HID,), jnp.float32),
            pltpu.SemaphoreType.DMA,
            pltpu.SemaphoreType.DMA,
        ],
    )(_body)
    return f(ids, segs, tok_tab, seg_tab, pos_tab, gamma, beta)


def kernel(input_ids, segment_ids, token_table, segment_table, position_table,
           gamma, beta):
    ids = input_ids.reshape(-1).astype(jnp.int32)
    segs = segment_ids.reshape(-1).astype(jnp.int32)
    out = _embed_ln(ids, segs, token_table, segment_table, position_table,
                    gamma, beta)
    return out.reshape(B, S, HID)
